# baseline (speedup 1.0000x reference)
import functools

import jax
import jax.numpy as jnp
from jax import lax
from jax.experimental import pallas as pl
from jax.experimental.pallas import tpu as pltpu
from jax.experimental.pallas import tpu_sc as plsc

D = 64
SCALE = float(D) ** 0.5
L = 16
NC = 2
NS = 16
NW = NC * NS
SB = 256
N_ROWS = 4096
N_COLS = 200
IH = N_ROWS // 128
SB_IH = SB // 128
N_SB = N_COLS * (N_ROWS // SB)
SB_PER_W = N_SB // NW
G16 = SB // L


def _sc_lookup_body(xt_hbm, t_hbm, out_hbm,
                    idx0, idx1, buf0, buf1, bt0, bt1,
                    x0, x1, g0, g1, s0, s1):
    idxv = (idx0, idx1)
    buf = (buf0, buf1)
    bufT = (bt0, bt1)
    xsem = (x0, x1)
    gsem = (g0, g1)
    ssem = (s0, s1)

    wid = lax.axis_index("s") * NC + lax.axis_index("c")
    sb_base = wid * SB_PER_W

    def fetch_idx(sb, b):
        j = sb // (N_ROWS // SB)
        i0 = (sb % (N_ROWS // SB)) * SB
        return pltpu.async_copy(
            xt_hbm.at[j, pl.ds(i0, SB)], idxv[b], xsem[b])

    def wait_idx(b):
        pltpu.make_async_copy(
            xt_hbm.at[0, pl.ds(0, SB)], idxv[b], xsem[b]).wait()

    def gather(b):
        return pltpu.async_copy(t_hbm.at[idxv[b]], buf[b], gsem[b])

    def wait_gather(b):
        pltpu.make_async_copy(t_hbm.at[idxv[b]], buf[b], gsem[b]).wait()

    def transform(b):
        def gbody(g, carry):
            rowv = jax.lax.iota(jnp.int32, L) + g * L
            g_hi = g // 8
            g_lo = g % 8
            for d in range(D):
                val = plsc.load_gather(
                    buf[b], [rowv, jnp.full((L,), d, jnp.int32)]) * SCALE
                bufT[b][d // 8, g_hi, d % 8, pl.ds(g_lo * L, L)] = val
            return carry
        lax.fori_loop(0, G16, gbody, 0)

    def scatter(sb, b):
        j = sb // (N_ROWS // SB)
        ihb = (sb % (N_ROWS // SB)) * SB_IH
        return pltpu.async_copy(
            bufT[b], out_hbm.at[j, :, pl.ds(ihb, SB_IH)], ssem[b])

    def wait_scatter(b):
        pltpu.make_async_copy(
            bufT[b], out_hbm.at[0, :, pl.ds(0, SB_IH)], ssem[b]).wait()

    fetch_idx(sb_base, 0)
    fetch_idx(sb_base + 1, 1)
    wait_idx(0)
    gather(0)

    def outer(g2, carry):
        for b in range(2):
            k = g2 * 2 + b
            sb = sb_base + k
            bn = 1 - b

            @pl.when(k + 1 < SB_PER_W)
            def _():
                wait_idx(bn)
                gather(bn)

            wait_gather(b)

            @pl.when(k >= 2)
            def _():
                wait_scatter(b)

            transform(b)
            scatter(sb, b)

            @pl.when(k + 2 < SB_PER_W)
            def _():
                fetch_idx(sb + 2, b)
        return carry

    lax.fori_loop(0, SB_PER_W // 2, outer, 0)

    wait_scatter(0)
    wait_scatter(1)


@functools.lru_cache(maxsize=None)
def _make_sc_lookup(V):
    mesh = plsc.VectorSubcoreMesh(core_axis_name="c", subcore_axis_name="s")
    return functools.partial(
        pl.kernel,
        mesh=mesh,
        out_type=jax.ShapeDtypeStruct((N_COLS, 8, IH, 8, 128), jnp.float32),
        scratch_types=[
            pltpu.VMEM((SB,), jnp.int32),
            pltpu.VMEM((SB,), jnp.int32),
            pltpu.VMEM((SB, D), jnp.float32),
            pltpu.VMEM((SB, D), jnp.float32),
            pltpu.VMEM((8, SB_IH, 8, 128), jnp.float32),
            pltpu.VMEM((8, SB_IH, 8, 128), jnp.float32),
            pltpu.SemaphoreType.DMA,
            pltpu.SemaphoreType.DMA,
            pltpu.SemaphoreType.DMA,
            pltpu.SemaphoreType.DMA,
            pltpu.SemaphoreType.DMA,
            pltpu.SemaphoreType.DMA,
        ],
        compiler_params=pltpu.CompilerParams(
            use_tc_tiling_on_sc=False, needs_layout_passes=False),
    )(_sc_lookup_body)


def kernel(x, table):
    V = table.shape[0]
    xt = x.T.astype(jnp.int32)
    out5d = _make_sc_lookup(V)(xt, table)
    out = out5d.transpose(2, 4, 0, 1, 3).reshape(N_ROWS, N_COLS, D)
    return out


# speedup vs baseline: 1.1758x; 1.1758x over previous
import functools

import jax
import jax.numpy as jnp
from jax import lax
from jax.experimental import pallas as pl
from jax.experimental.pallas import tpu as pltpu
from jax.experimental.pallas import tpu_sc as plsc

D = 64
SCALE = float(D) ** 0.5
L = 16
NC = 2
NS = 16
NW = NC * NS
SB = 256
N_ROWS = 4096
N_COLS = 200
IH = N_ROWS // 128
SB_IH = SB // 128
N_SB = N_COLS * (N_ROWS // SB)
SB_PER_W = N_SB // NW
G16 = SB // L


def _sc_lookup_body(xt_hbm, t_hbm, out_hbm,
                    idx0, idx1, buf0, buf1, bufP, bt0, bt1,
                    x0, x1, g0, g1, s0, s1):
    idxv = (idx0, idx1)
    buf = (buf0, buf1)
    bufT = (bt0, bt1)
    xsem = (x0, x1)
    gsem = (g0, g1)
    ssem = (s0, s1)

    wid = lax.axis_index("s") * NC + lax.axis_index("c")
    sb_base = wid * SB_PER_W

    def fetch_idx(sb, b):
        j = sb // (N_ROWS // SB)
        i0 = (sb % (N_ROWS // SB)) * SB
        return pltpu.async_copy(
            xt_hbm.at[j, pl.ds(i0, SB)], idxv[b], xsem[b])

    def wait_idx(b):
        pltpu.make_async_copy(
            xt_hbm.at[0, pl.ds(0, SB)], idxv[b], xsem[b]).wait()

    def gather(b):
        return pltpu.async_copy(t_hbm.at[idxv[b]], buf[b], gsem[b])

    def wait_gather(b):
        pltpu.make_async_copy(t_hbm.at[idxv[b]], buf[b], gsem[b]).wait()

    def transform(b):
        def prow(i, carry):
            for c in range(D // L):
                bufP[i, pl.ds(c * L, L)] = buf[b][i, pl.ds(c * L, L)]
            return carry
        lax.fori_loop(0, SB, prow, 0)

        def gbody(g, carry):
            rowv = jax.lax.iota(jnp.int32, L) + g * L
            g_hi = g // 8
            g_lo = g % 8
            for d in range(D):
                val = plsc.load_gather(
                    bufP, [rowv, jnp.full((L,), d, jnp.int32)]) * SCALE
                bufT[b][d // 8, g_hi, d % 8, pl.ds(g_lo * L, L)] = val
            return carry
        lax.fori_loop(0, G16, gbody, 0)

    def scatter(sb, b):
        j = sb // (N_ROWS // SB)
        ihb = (sb % (N_ROWS // SB)) * SB_IH
        return pltpu.async_copy(
            bufT[b], out_hbm.at[j, :, pl.ds(ihb, SB_IH)], ssem[b])

    def wait_scatter(b):
        pltpu.make_async_copy(
            bufT[b], out_hbm.at[0, :, pl.ds(0, SB_IH)], ssem[b]).wait()

    fetch_idx(sb_base, 0)
    fetch_idx(sb_base + 1, 1)
    wait_idx(0)
    gather(0)

    def outer(g2, carry):
        for b in range(2):
            k = g2 * 2 + b
            sb = sb_base + k
            bn = 1 - b

            @pl.when(k + 1 < SB_PER_W)
            def _():
                wait_idx(bn)
                gather(bn)

            wait_gather(b)

            @pl.when(k >= 2)
            def _():
                wait_scatter(b)

            transform(b)
            scatter(sb, b)

            @pl.when(k + 2 < SB_PER_W)
            def _():
                fetch_idx(sb + 2, b)
        return carry

    lax.fori_loop(0, SB_PER_W // 2, outer, 0)

    wait_scatter(0)
    wait_scatter(1)


@functools.lru_cache(maxsize=None)
def _make_sc_lookup(V):
    mesh = plsc.VectorSubcoreMesh(core_axis_name="c", subcore_axis_name="s")
    return functools.partial(
        pl.kernel,
        mesh=mesh,
        out_type=jax.ShapeDtypeStruct((N_COLS, 8, IH, 8, 128), jnp.float32),
        scratch_types=[
            pltpu.VMEM((SB,), jnp.int32),
            pltpu.VMEM((SB,), jnp.int32),
            pltpu.VMEM((SB, D), jnp.float32),
            pltpu.VMEM((SB, D), jnp.float32),
            pltpu.VMEM((SB, D + 1), jnp.float32),
            pltpu.VMEM((8, SB_IH, 8, 128), jnp.float32),
            pltpu.VMEM((8, SB_IH, 8, 128), jnp.float32),
            pltpu.SemaphoreType.DMA,
            pltpu.SemaphoreType.DMA,
            pltpu.SemaphoreType.DMA,
            pltpu.SemaphoreType.DMA,
            pltpu.SemaphoreType.DMA,
            pltpu.SemaphoreType.DMA,
        ],
        compiler_params=pltpu.CompilerParams(
            use_tc_tiling_on_sc=False, needs_layout_passes=False),
    )(_sc_lookup_body)


def kernel(x, table):
    V = table.shape[0]
    xt = x.T.astype(jnp.int32)
    out5d = _make_sc_lookup(V)(xt, table)
    out = out5d.transpose(2, 4, 0, 1, 3).reshape(N_ROWS, N_COLS, D)
    return out


# speedup vs baseline: 1.5836x; 1.3468x over previous
import functools

import jax
import jax.numpy as jnp
from jax import lax
from jax.experimental import pallas as pl
from jax.experimental.pallas import tpu as pltpu
from jax.experimental.pallas import tpu_sc as plsc

D = 64
SCALE = float(D) ** 0.5
L = 16
NC = 2
NS = 16
NW = NC * NS
N_ROWS = 4096
N_COLS = 200
ROWS_PER_W = N_ROWS // NW


def _sc_lookup_body(xf_hbm, t_hbm, out_hbm,
                    idx0, idx1, buf0, buf1,
                    x0, x1, g0, g1, s0, s1):
    idxv = (idx0, idx1)
    buf = (buf0, buf1)
    xsem = (x0, x1)
    gsem = (g0, g1)
    ssem = (s0, s1)

    wid = lax.axis_index("s") * NC + lax.axis_index("c")
    row_base = wid * ROWS_PER_W

    def fetch_idx(row, b):
        return pltpu.async_copy(
            xf_hbm.at[pl.ds(row * N_COLS, N_COLS)], idxv[b], xsem[b])

    def wait_idx(b):
        pltpu.make_async_copy(
            xf_hbm.at[pl.ds(0, N_COLS)], idxv[b], xsem[b]).wait()

    def gather(b):
        return pltpu.async_copy(t_hbm.at[idxv[b]], buf[b], gsem[b])

    def wait_gather(b):
        pltpu.make_async_copy(t_hbm.at[idxv[b]], buf[b], gsem[b]).wait()

    def scale(b):
        def body(i, carry):
            for c in range(D // L):
                sl = pl.ds(c * L, L)
                buf[b][i, sl] = buf[b][i, sl] * SCALE
            return carry
        lax.fori_loop(0, N_COLS, body, 0, unroll=4)

    def scatter(row, b):
        return pltpu.async_copy(buf[b], out_hbm.at[row], ssem[b])

    def wait_scatter(b):
        pltpu.make_async_copy(buf[b], out_hbm.at[0], ssem[b]).wait()

    fetch_idx(row_base, 0)
    fetch_idx(row_base + 1, 1)
    wait_idx(0)
    gather(0)

    def outer(g2, carry):
        for b in range(2):
            k = g2 * 2 + b
            row = row_base + k
            bn = 1 - b

            @pl.when(k + 1 < ROWS_PER_W)
            def _():
                wait_idx(bn)

                @pl.when(k >= 1)
                def _():
                    wait_scatter(bn)
                gather(bn)

            wait_gather(b)
            scale(b)
            scatter(row, b)

            @pl.when(k + 2 < ROWS_PER_W)
            def _():
                fetch_idx(row + 2, b)
        return carry

    lax.fori_loop(0, ROWS_PER_W // 2, outer, 0)

    wait_scatter(0)
    wait_scatter(1)


@functools.lru_cache(maxsize=None)
def _make_sc_lookup(V):
    mesh = plsc.VectorSubcoreMesh(core_axis_name="c", subcore_axis_name="s")
    return functools.partial(
        pl.kernel,
        mesh=mesh,
        out_type=jax.ShapeDtypeStruct((N_ROWS, N_COLS, D), jnp.float32),
        scratch_types=[
            pltpu.VMEM((N_COLS,), jnp.int32),
            pltpu.VMEM((N_COLS,), jnp.int32),
            pltpu.VMEM((N_COLS, D), jnp.float32),
            pltpu.VMEM((N_COLS, D), jnp.float32),
            pltpu.SemaphoreType.DMA,
            pltpu.SemaphoreType.DMA,
            pltpu.SemaphoreType.DMA,
            pltpu.SemaphoreType.DMA,
            pltpu.SemaphoreType.DMA,
            pltpu.SemaphoreType.DMA,
        ],
        compiler_params=pltpu.CompilerParams(
            use_tc_tiling_on_sc=False, needs_layout_passes=False),
    )(_sc_lookup_body)


def kernel(x, table):
    V = table.shape[0]
    xf = x.reshape(-1).astype(jnp.int32)
    return _make_sc_lookup(V)(xf, table)


# speedup vs baseline: 1.6140x; 1.0192x over previous
import functools

import jax
import jax.numpy as jnp
from jax import lax
from jax.experimental import pallas as pl
from jax.experimental.pallas import tpu as pltpu
from jax.experimental.pallas import tpu_sc as plsc

D = 64
SCALE = float(D) ** 0.5
L = 16
NC = 2
NS = 16
NW = NC * NS
CHUNK = 640


@functools.lru_cache(maxsize=None)
def _make_sc_lookup(B, V):
    assert B % (NW * CHUNK) == 0
    b_per_w = B // NW
    n_chunks = b_per_w // CHUNK
    mesh = plsc.VectorSubcoreMesh(core_axis_name="c", subcore_axis_name="s")

    @functools.partial(
        pl.kernel,
        mesh=mesh,
        out_type=jax.ShapeDtypeStruct((B, D), jnp.float32),
        scratch_types=[
            pltpu.VMEM((b_per_w,), jnp.int32),
            pltpu.VMEM((CHUNK, D), jnp.float32),
            pltpu.VMEM((CHUNK, D), jnp.float32),
            pltpu.SemaphoreType.DMA,
            pltpu.SemaphoreType.DMA,
            pltpu.SemaphoreType.DMA,
            pltpu.SemaphoreType.DMA,
        ],
        compiler_params=pltpu.CompilerParams(use_tc_tiling_on_sc=False),
    )
    def sc_lookup(x_hbm, table_hbm, out_hbm, idx_v, rows0, rows1,
                  g0, g1, s0, s1):
        wid = lax.axis_index("s") * NC + lax.axis_index("c")
        base = wid * b_per_w
        rows = (rows0, rows1)
        gsem = (g0, g1)
        ssem = (s0, s1)

        pltpu.sync_copy(x_hbm.at[pl.ds(base, b_per_w)], idx_v)

        def gather(c, b):
            pltpu.async_copy(
                table_hbm.at[idx_v.at[pl.ds(c * CHUNK, CHUNK)]],
                rows[b], gsem[b])

        def scale(b):
            def body(i, carry):
                for j in range(D // L):
                    sl = pl.ds(j * L, L)
                    rows[b][i, sl] = rows[b][i, sl] * SCALE
                return carry
            lax.fori_loop(0, CHUNK, body, 0, unroll=8)

        def scatter(c, b):
            return pltpu.async_copy(
                rows[b], out_hbm.at[pl.ds(base + c * CHUNK, CHUNK)], ssem[b])

        def wait_gather(b):
            pltpu.make_async_copy(
                table_hbm.at[idx_v.at[pl.ds(0, CHUNK)]], rows[b],
                gsem[b]).wait()

        def wait_scatter(b):
            pltpu.make_async_copy(
                rows[b], out_hbm.at[pl.ds(base, CHUNK)], ssem[b]).wait()

        gather(0, 0)

        def outer(g, carry):
            for b in range(2):
                c = g + b
                bn = 1 - b

                @pl.when((c >= 1) & (c + 1 < n_chunks))
                def _():
                    wait_scatter(bn)

                @pl.when(c + 1 < n_chunks)
                def _():
                    gather(c + 1, bn)

                wait_gather(b)
                scale(b)
                scatter(c, b)
            return carry

        lax.fori_loop(0, n_chunks // 2, lambda g, cc: outer(g * 2, cc), 0)

        wait_scatter(0)
        wait_scatter(1)

    return sc_lookup


def kernel(x, table):
    xf = x.reshape(-1).astype(jnp.int32)
    out = _make_sc_lookup(xf.shape[0], table.shape[0])(xf, table)
    return out.reshape(x.shape + (D,))
